# baseline (speedup 1.0000x reference)
import functools

import jax
import jax.numpy as jnp
from jax import lax
from jax.experimental import pallas as pl
from jax.experimental.pallas import tpu as pltpu
from jax.experimental.pallas import tpu_sc as plsc

B = 16
P = 1024
K = 20
NP = B * P
E = NP * K
EPS = 1e-5
F32 = jnp.float32

TPK = 512
TPE = 4096
NBP = NP // TPE
TPP = 512

_NC, _NS = 2, 16
_NW = _NC * _NS


def _popmin(d2, iota, nkeep):
    n = float(d2.shape[1])
    cols = []
    for _ in range(nkeep):
        m = jnp.min(d2, axis=1, keepdims=True)
        cand = jnp.where(d2 == m, iota, n)
        j = jnp.min(cand, axis=1, keepdims=True)
        cols.append(j.astype(jnp.int32))
        d2 = jnp.where(iota == j, jnp.inf, d2)
    return cols


BF = jnp.bfloat16


def _dot1x(a, b):
    return jnp.dot(a.astype(BF), b.astype(BF), preferred_element_type=F32)


def _knn1_body(posp_ref, post_ref, posb_ref, wa_ref, wb_ref, b1_ref,
               xjp_ref, st_ref):
    g = pl.program_id(0) * (P // TPK) + pl.program_id(1)
    x = posp_ref[...]
    xt = post_ref[0]
    pb = posb_ref[...]
    hi = pb.astype(BF)
    r1 = pb - hi.astype(F32)
    mid = r1.astype(BF)
    lo = (r1 - mid.astype(F32)).astype(BF)
    tab = jnp.concatenate([hi, mid, lo], axis=1)
    sq_r = jnp.sum(x * x, axis=1, keepdims=True)
    sq_c = jnp.sum(xt * xt, axis=0, keepdims=True)
    d2 = sq_r + sq_c - 2.0 * _dot1x(x, xt)
    iota = lax.broadcasted_iota(jnp.int32, (TPK, P), 1).astype(F32)
    wb = wb_ref[...].astype(BF)
    h0 = jnp.dot(x.astype(BF), wa_ref[...].astype(BF),
                 preferred_element_type=F32) + b1_ref[...]
    ssum = jnp.zeros((1, 64), F32)
    ssq = jnp.zeros((1, 64), F32)
    xjs = []
    for _ in range(K):
        m = jnp.min(d2, axis=1, keepdims=True)
        cand = jnp.where(d2 == m, iota, float(P))
        j = jnp.min(cand, axis=1, keepdims=True)
        sel = iota == j
        xq = jnp.dot(sel.astype(BF), tab, preferred_element_type=F32)
        xj = xq[:, 0:16] + xq[:, 16:32] + xq[:, 32:48]
        xjs.append(xj)
        h1k = h0 + jnp.dot((xj - x).astype(BF), wb, preferred_element_type=F32)
        ssum = ssum + jnp.sum(h1k, axis=0, keepdims=True)
        ssq = ssq + jnp.sum(h1k * h1k, axis=0, keepdims=True)
        d2 = jnp.where(sel, jnp.inf, d2)
    xjp_ref[...] = jnp.concatenate(xjs, axis=1)
    st = jnp.concatenate([ssum, ssq], axis=0)

    @pl.when(g == 0)
    def _():
        st_ref[...] = st

    @pl.when(g != 0)
    def _():
        st_ref[...] = st_ref[...] + st


def _knn2_body(x_ref, xt_ref, idx_ref):
    b = pl.program_id(0)
    p = pl.program_id(1)
    x = x_ref[0]
    xt = xt_ref[0]
    sq_r = jnp.sum(x * x, axis=1, keepdims=True)
    sq_c = jnp.sum(xt * xt, axis=0, keepdims=True)
    d2 = sq_r + sq_c - 2.0 * _dot1x(x, xt)
    iota = lax.broadcasted_iota(jnp.int32, (TPK, P), 1).astype(F32)
    cols = _popmin(d2, iota, K)
    idx_ref[...] = jnp.concatenate(cols, axis=1) + b * P


G4 = 4


def _sc_gather_max(y, idx):
    pw = NP // _NW
    ngrp = pw // G4
    gi = G4 * K
    mesh = plsc.VectorSubcoreMesh(core_axis_name="c", subcore_axis_name="s")

    @functools.partial(
        pl.kernel, mesh=mesh,
        out_type=jax.ShapeDtypeStruct((NP, 128), F32),
        scratch_types=[pltpu.VMEM((pw * K,), jnp.int32),
                       pltpu.VMEM((gi, 128), F32),
                       pltpu.VMEM((gi, 128), F32),
                       pltpu.VMEM((G4, 128), F32),
                       pltpu.SemaphoreType.DMA,
                       pltpu.SemaphoreType.DMA],
    )
    def run(y_hbm, idx_hbm, out_hbm, idx_all, rows0, rows1, out_v, sem0, sem1):
        wid = lax.axis_index("s") * _NC + lax.axis_index("c")
        base = wid * pw
        pltpu.sync_copy(idx_hbm.at[pl.ds(base * K, pw * K)], idx_all)
        pltpu.async_copy(y_hbm.at[idx_all.at[pl.ds(0, gi)]], rows0, sem0)
        pltpu.async_copy(y_hbm.at[idx_all.at[pl.ds(gi, gi)]], rows1, sem1)

        def half(g, rows_v, sem):
            pltpu.make_async_copy(y_hbm.at[idx_all.at[pl.ds(0, gi)]],
                                  rows_v, sem).wait()
            for i in range(G4):
                for c in range(8):
                    v = rows_v[i * K, pl.ds(c * 16, 16)]
                    for r in range(1, K):
                        v = jnp.maximum(v, rows_v[i * K + r, pl.ds(c * 16, 16)])
                    out_v[i, pl.ds(c * 16, 16)] = v
            pltpu.sync_copy(out_v, out_hbm.at[pl.ds(base + g * G4, G4)])
            nxt = g + 2

            @pl.when(nxt < ngrp)
            def _():
                pltpu.async_copy(y_hbm.at[idx_all.at[pl.ds(nxt * gi, gi)]],
                                 rows_v, sem)

        def body(gg, carry):
            half(2 * gg, rows0, sem0)
            half(2 * gg + 1, rows1, sem1)
            return carry

        lax.fori_loop(0, ngrp // 2, body, 0)

    return run(y, idx)


def _stats_update(st_ref, h, g):
    st = jnp.concatenate([jnp.sum(h, axis=0, keepdims=True),
                          jnp.sum(h * h, axis=0, keepdims=True)], axis=0)

    @pl.when(g == 0)
    def _():
        st_ref[...] = st

    @pl.when(g != 0)
    def _():
        st_ref[...] = st_ref[...] + st


def _norm_consts(st):
    mu = st[0:1] * (1.0 / E)
    var = st[1:2] * (1.0 / E) - mu * mu
    return mu, lax.rsqrt(var + EPS)


def _h1k(xi, xjp, wb, hi, k):
    xj = xjp[:, k * 16:(k + 1) * 16]
    return hi + jnp.dot((xj - xi).astype(BF), wb, preferred_element_type=F32)


def _edge2_body(xi_ref, xjp_ref, st1_ref, wa_ref, wb_ref, b1_ref,
                w2_ref, b2_ref, st_ref):
    g = pl.program_id(0)
    xi = xi_ref[...]
    xjp = xjp_ref[...]
    wb = wb_ref[...].astype(BF)
    hi = jnp.dot(xi.astype(BF), wa_ref[...].astype(BF),
                 preferred_element_type=F32) + b1_ref[...]
    mu, rs = _norm_consts(st1_ref[...])
    ssum = jnp.zeros((1, 64), F32)
    ssq = jnp.zeros((1, 64), F32)
    for k in range(K):
        hn = jnp.maximum((_h1k(xi, xjp, wb, hi, k) - mu) * rs, 0.0)
        h2k = _dot1x(hn, w2_ref[...]) + b2_ref[...]
        ssum = ssum + jnp.sum(h2k, axis=0, keepdims=True)
        ssq = ssq + jnp.sum(h2k * h2k, axis=0, keepdims=True)
    st = jnp.concatenate([ssum, ssq], axis=0)

    @pl.when(g == 0)
    def _():
        st_ref[...] = st

    @pl.when(g != 0)
    def _():
        st_ref[...] = st_ref[...] + st


def _edge3_body(xi_ref, xjp_ref, st1_ref, st2_ref, wa_ref, wb_ref, b1_ref,
                w2_ref, b2_ref, w3_ref, b3_ref, w4b_ref, w4d_ref, b4_ref,
                x1_ref, y_ref, u_ref):
    xi = xi_ref[...]
    xjp = xjp_ref[...]
    wb = wb_ref[...].astype(BF)
    hi = jnp.dot(xi.astype(BF), wa_ref[...].astype(BF),
                 preferred_element_type=F32) + b1_ref[...]
    mu1, rs1 = _norm_consts(st1_ref[...])
    mu2, rs2 = _norm_consts(st2_ref[...])
    acc = jnp.full((TPP, 64), -jnp.inf, F32)
    for k in range(K):
        hn = jnp.maximum((_h1k(xi, xjp, wb, hi, k) - mu1) * rs1, 0.0)
        h2k = _dot1x(hn, w2_ref[...]) + b2_ref[...]
        hn2 = jnp.maximum((h2k - mu2) * rs2, 0.0)
        v = _dot1x(hn2, w3_ref[...]) + b3_ref[...]
        acc = jnp.maximum(acc, v)
    x1_ref[...] = acc
    y_ref[...] = _dot1x(acc, w4b_ref[...])
    u_ref[...] = _dot1x(acc, w4d_ref[...]) + b4_ref[...]


def _pool_body(x1_ref, u_ref, m_ref, w5a_ref, w5b_ref, b5_ref, out_ref):
    p = pl.program_id(1)
    t = (_dot1x(x1_ref[...], w5a_ref[...])
         + _dot1x(u_ref[...] + m_ref[...], w5b_ref[...])
         + b5_ref[...])
    v = jnp.broadcast_to(jnp.max(t, axis=0, keepdims=True), (8, 1024))[None]

    @pl.when(p == 0)
    def _():
        out_ref[...] = v

    @pl.when(p != 0)
    def _():
        out_ref[...] = jnp.maximum(out_ref[...], v)


def _bn_relu_rows(h):
    mu = jnp.mean(h, axis=0, keepdims=True)
    var = jnp.mean((h - mu) ** 2, axis=0, keepdims=True)
    return jnp.maximum((h - mu) * lax.rsqrt(var + EPS), 0.0)


def _head_body(z_ref, w6_ref, b6_ref, w7_ref, b7_ref, w8_ref, b8_ref, o_ref):
    h = _dot1x(z_ref[...], w6_ref[...]) + b6_ref[...]
    h = _bn_relu_rows(h)
    h = _dot1x(h, w7_ref[...]) + b7_ref[...]
    h = _bn_relu_rows(h)
    h = _dot1x(h, w8_ref[...]) + b8_ref[...]
    mx = jnp.max(h, axis=1, keepdims=True)
    e = jnp.exp(h - mx)
    o_ref[...] = h - mx - jnp.log(jnp.sum(e, axis=1, keepdims=True))


def kernel(pos, batch, W1, b1, W2, b2, W3, b3, W4, b4, W5, b5, W6, b6, W7, b7, W8, b8):
    del batch
    posp = jnp.pad(pos, ((0, 0), (0, 13)))
    post = jnp.pad(pos.reshape(B, P, 3).transpose(0, 2, 1),
                   ((0, 0), (0, 13), (0, 0)))
    w1a = jnp.pad(W1[0:3], ((0, 13), (0, 0)))
    w1b = jnp.pad(W1[3:6], ((0, 13), (0, 0)))
    w4a, w4b = W4[:64], W4[64:]
    w4d = w4a - w4b
    w5a, w5b = W5[:64], W5[64:]

    nb = P // TPK

    b1r = b1.reshape(1, 64)
    b2r = b2.reshape(1, 64)

    xjp, st1 = pl.pallas_call(
        _knn1_body, grid=(B, nb),
        in_specs=[pl.BlockSpec((TPK, 16), lambda b, p: (b * nb + p, 0)),
                  pl.BlockSpec((1, 16, P), lambda b, p: (b, 0, 0)),
                  pl.BlockSpec((P, 16), lambda b, p: (b, 0)),
                  pl.BlockSpec((16, 64), lambda b, p: (0, 0)),
                  pl.BlockSpec((16, 64), lambda b, p: (0, 0)),
                  pl.BlockSpec((1, 64), lambda b, p: (0, 0))],
        out_specs=[pl.BlockSpec((TPK, K * 16), lambda b, p: (b * nb + p, 0)),
                   pl.BlockSpec((2, 64), lambda b, p: (0, 0))],
        out_shape=[jax.ShapeDtypeStruct((NP, K * 16), F32),
                   jax.ShapeDtypeStruct((2, 64), F32)],
    )(posp, post, posp, w1a, w1b, b1r)

    st2 = pl.pallas_call(
        _edge2_body, grid=(NP // TPP,),
        in_specs=[pl.BlockSpec((TPP, 16), lambda p: (p, 0)),
                  pl.BlockSpec((TPP, K * 16), lambda p: (p, 0)),
                  pl.BlockSpec((2, 64), lambda p: (0, 0)),
                  pl.BlockSpec((16, 64), lambda p: (0, 0)),
                  pl.BlockSpec((16, 64), lambda p: (0, 0)),
                  pl.BlockSpec((1, 64), lambda p: (0, 0)),
                  pl.BlockSpec((64, 64), lambda p: (0, 0)),
                  pl.BlockSpec((1, 64), lambda p: (0, 0))],
        out_specs=pl.BlockSpec((2, 64), lambda p: (0, 0)),
        out_shape=jax.ShapeDtypeStruct((2, 64), F32),
    )(posp, xjp, st1, w1a, w1b, b1r, W2, b2r)

    x1, y, u = pl.pallas_call(
        _edge3_body, grid=(NP // TPP,),
        in_specs=[pl.BlockSpec((TPP, 16), lambda p: (p, 0)),
                  pl.BlockSpec((TPP, K * 16), lambda p: (p, 0)),
                  pl.BlockSpec((2, 64), lambda p: (0, 0)),
                  pl.BlockSpec((2, 64), lambda p: (0, 0)),
                  pl.BlockSpec((16, 64), lambda p: (0, 0)),
                  pl.BlockSpec((16, 64), lambda p: (0, 0)),
                  pl.BlockSpec((1, 64), lambda p: (0, 0)),
                  pl.BlockSpec((64, 64), lambda p: (0, 0)),
                  pl.BlockSpec((1, 64), lambda p: (0, 0)),
                  pl.BlockSpec((64, 64), lambda p: (0, 0)),
                  pl.BlockSpec((1, 64), lambda p: (0, 0)),
                  pl.BlockSpec((64, 128), lambda p: (0, 0)),
                  pl.BlockSpec((64, 128), lambda p: (0, 0)),
                  pl.BlockSpec((1, 128), lambda p: (0, 0))],
        out_specs=[pl.BlockSpec((TPP, 64), lambda p: (p, 0)),
                   pl.BlockSpec((TPP, 128), lambda p: (p, 0)),
                   pl.BlockSpec((TPP, 128), lambda p: (p, 0))],
        out_shape=[jax.ShapeDtypeStruct((NP, 64), F32),
                   jax.ShapeDtypeStruct((NP, 128), F32),
                   jax.ShapeDtypeStruct((NP, 128), F32)],
    )(posp, xjp, st1, st2, w1a, w1b, b1r, W2, b2r, W3, b3.reshape(1, 64),
      w4b, w4d, b4.reshape(1, 128))

    x1v = x1.reshape(B, P, 64)
    x1t = x1v.transpose(0, 2, 1)
    idx2 = pl.pallas_call(
        _knn2_body, grid=(B, nb),
        in_specs=[pl.BlockSpec((1, TPK, 64), lambda b, p: (b, p, 0)),
                  pl.BlockSpec((1, 64, P), lambda b, p: (b, 0, 0))],
        out_specs=pl.BlockSpec((TPK, K), lambda b, p: (b * nb + p, 0)),
        out_shape=jax.ShapeDtypeStruct((NP, K), jnp.int32),
    )(x1v, x1t)

    m = _sc_gather_max(y, idx2.reshape(-1))

    pooled = pl.pallas_call(
        _pool_body, grid=(B, P // TPP),
        in_specs=[pl.BlockSpec((TPP, 64), lambda b, p: (b * (P // TPP) + p, 0)),
                  pl.BlockSpec((TPP, 128), lambda b, p: (b * (P // TPP) + p, 0)),
                  pl.BlockSpec((TPP, 128), lambda b, p: (b * (P // TPP) + p, 0)),
                  pl.BlockSpec((64, 1024), lambda b, p: (0, 0)),
                  pl.BlockSpec((128, 1024), lambda b, p: (0, 0)),
                  pl.BlockSpec((1, 1024), lambda b, p: (0, 0))],
        out_specs=pl.BlockSpec((1, 8, 1024), lambda b, p: (b, 0, 0)),
        out_shape=jax.ShapeDtypeStruct((B, 8, 1024), F32),
    )(x1, u, m, w5a, w5b, b5.reshape(1, 1024))
    pooled = pooled[:, 0, :]

    out = pl.pallas_call(
        _head_body,
        in_specs=[pl.BlockSpec((B, 1024), lambda: (0, 0)),
                  pl.BlockSpec((1024, 512), lambda: (0, 0)),
                  pl.BlockSpec((1, 512), lambda: (0, 0)),
                  pl.BlockSpec((512, 256), lambda: (0, 0)),
                  pl.BlockSpec((1, 256), lambda: (0, 0)),
                  pl.BlockSpec((256, 40), lambda: (0, 0)),
                  pl.BlockSpec((1, 40), lambda: (0, 0))],
        out_specs=pl.BlockSpec((B, 40), lambda: (0, 0)),
        out_shape=jax.ShapeDtypeStruct((B, 40), F32),
    )(pooled, W6, b6.reshape(1, 512), W7, b7.reshape(1, 256),
      W8, b8.reshape(1, 40))
    return out


# speedup vs baseline: 1.0823x; 1.0823x over previous
import functools

import jax
import jax.numpy as jnp
from jax import lax
from jax.experimental import pallas as pl
from jax.experimental.pallas import tpu as pltpu
from jax.experimental.pallas import tpu_sc as plsc

B = 16
P = 1024
K = 20
NP = B * P
E = NP * K
EPS = 1e-5
F32 = jnp.float32

TPK = 512
TPE = 4096
NBP = NP // TPE
TPP = 512

_NC, _NS = 2, 16
_NW = _NC * _NS


def _popmin(d2, iota, nkeep):
    n = float(d2.shape[1])
    cols = []
    for _ in range(nkeep):
        m = jnp.min(d2, axis=1, keepdims=True)
        cand = jnp.where(d2 == m, iota, n)
        j = jnp.min(cand, axis=1, keepdims=True)
        cols.append(j.astype(jnp.int32))
        d2 = jnp.where(iota == j, jnp.inf, d2)
    return cols


BF = jnp.bfloat16


def _dot1x(a, b):
    return jnp.dot(a.astype(BF), b.astype(BF), preferred_element_type=F32)


def _knn1_body(posp_ref, post_ref, posb_ref, xjp_ref):
    x = posp_ref[...]
    xt = post_ref[0]
    pb = posb_ref[...]
    hi = pb.astype(BF)
    r1 = pb - hi.astype(F32)
    mid = r1.astype(BF)
    lo = (r1 - mid.astype(F32)).astype(BF)
    tab = jnp.concatenate([hi, mid, lo], axis=1)
    sq_r = jnp.sum(x * x, axis=1, keepdims=True)
    sq_c = jnp.sum(xt * xt, axis=0, keepdims=True)
    d2 = sq_r + sq_c - 2.0 * _dot1x(x, xt)
    iota = lax.broadcasted_iota(jnp.int32, (TPK, P), 1).astype(F32)
    xjs = []
    for _ in range(K):
        m = jnp.min(d2, axis=1, keepdims=True)
        cand = jnp.where(d2 == m, iota, float(P))
        j = jnp.min(cand, axis=1, keepdims=True)
        sel = iota == j
        xq = jnp.dot(sel.astype(BF), tab, preferred_element_type=F32)
        xjs.append(xq[:, 0:16] + xq[:, 16:32] + xq[:, 32:48])
        d2 = jnp.where(sel, jnp.inf, d2)
    xjp_ref[...] = jnp.concatenate(xjs, axis=1)


def _edge1_body(xi_ref, xjp_ref, wa_ref, wb_ref, b1_ref, st_ref):
    g = pl.program_id(0)
    xi = xi_ref[...]
    xjp = xjp_ref[...]
    wb = wb_ref[...].astype(BF)
    hi = jnp.dot(xi.astype(BF), wa_ref[...].astype(BF),
                 preferred_element_type=F32) + b1_ref[...]
    ssum = jnp.zeros((1, 64), F32)
    ssq = jnp.zeros((1, 64), F32)
    for k in range(K):
        h1k = _h1k(xi, xjp, wb, hi, k)
        ssum = ssum + jnp.sum(h1k, axis=0, keepdims=True)
        ssq = ssq + jnp.sum(h1k * h1k, axis=0, keepdims=True)
    st = jnp.concatenate([ssum, ssq], axis=0)

    @pl.when(g == 0)
    def _():
        st_ref[...] = st

    @pl.when(g != 0)
    def _():
        st_ref[...] = st_ref[...] + st


def _knn2_body(x_ref, xt_ref, idx_ref):
    b = pl.program_id(0)
    p = pl.program_id(1)
    x = x_ref[0]
    xt = xt_ref[0]
    sq_r = jnp.sum(x * x, axis=1, keepdims=True)
    sq_c = jnp.sum(xt * xt, axis=0, keepdims=True)
    d2 = sq_r + sq_c - 2.0 * _dot1x(x, xt)
    iota = lax.broadcasted_iota(jnp.int32, (TPK, P), 1).astype(F32)
    cols = _popmin(d2, iota, K)
    idx_ref[...] = jnp.concatenate(cols, axis=1) + b * P


G4 = 4


def _sc_gather_max(y, idx):
    pw = NP // _NW
    ngrp = pw // G4
    gi = G4 * K
    mesh = plsc.VectorSubcoreMesh(core_axis_name="c", subcore_axis_name="s")

    @functools.partial(
        pl.kernel, mesh=mesh,
        out_type=jax.ShapeDtypeStruct((NP, 128), F32),
        scratch_types=[pltpu.VMEM((pw * K,), jnp.int32),
                       pltpu.VMEM((gi, 128), F32),
                       pltpu.VMEM((gi, 128), F32),
                       pltpu.VMEM((G4, 128), F32),
                       pltpu.SemaphoreType.DMA,
                       pltpu.SemaphoreType.DMA],
    )
    def run(y_hbm, idx_hbm, out_hbm, idx_all, rows0, rows1, out_v, sem0, sem1):
        wid = lax.axis_index("s") * _NC + lax.axis_index("c")
        base = wid * pw
        pltpu.sync_copy(idx_hbm.at[pl.ds(base * K, pw * K)], idx_all)
        pltpu.async_copy(y_hbm.at[idx_all.at[pl.ds(0, gi)]], rows0, sem0)
        pltpu.async_copy(y_hbm.at[idx_all.at[pl.ds(gi, gi)]], rows1, sem1)

        def half(g, rows_v, sem):
            pltpu.make_async_copy(y_hbm.at[idx_all.at[pl.ds(0, gi)]],
                                  rows_v, sem).wait()
            for i in range(G4):
                for c in range(8):
                    v = rows_v[i * K, pl.ds(c * 16, 16)]
                    for r in range(1, K):
                        v = jnp.maximum(v, rows_v[i * K + r, pl.ds(c * 16, 16)])
                    out_v[i, pl.ds(c * 16, 16)] = v
            pltpu.sync_copy(out_v, out_hbm.at[pl.ds(base + g * G4, G4)])
            nxt = g + 2

            @pl.when(nxt < ngrp)
            def _():
                pltpu.async_copy(y_hbm.at[idx_all.at[pl.ds(nxt * gi, gi)]],
                                 rows_v, sem)

        def body(gg, carry):
            half(2 * gg, rows0, sem0)
            half(2 * gg + 1, rows1, sem1)
            return carry

        lax.fori_loop(0, ngrp // 2, body, 0)

    return run(y, idx)


def _stats_update(st_ref, h, g):
    st = jnp.concatenate([jnp.sum(h, axis=0, keepdims=True),
                          jnp.sum(h * h, axis=0, keepdims=True)], axis=0)

    @pl.when(g == 0)
    def _():
        st_ref[...] = st

    @pl.when(g != 0)
    def _():
        st_ref[...] = st_ref[...] + st


def _norm_consts(st):
    mu = st[0:1] * (1.0 / E)
    var = st[1:2] * (1.0 / E) - mu * mu
    return mu, lax.rsqrt(var + EPS)


def _h1k(xi, xjp, wb, hi, k):
    xj = xjp[:, k * 16:(k + 1) * 16]
    return hi + jnp.dot((xj - xi).astype(BF), wb, preferred_element_type=F32)


def _edge2_body(xi_ref, xjp_ref, st1_ref, wa_ref, wb_ref, b1_ref,
                w2_ref, b2_ref, st_ref):
    g = pl.program_id(0)
    xi = xi_ref[...]
    xjp = xjp_ref[...]
    wb = wb_ref[...].astype(BF)
    hi = jnp.dot(xi.astype(BF), wa_ref[...].astype(BF),
                 preferred_element_type=F32) + b1_ref[...]
    mu, rs = _norm_consts(st1_ref[...])
    ssum = jnp.zeros((1, 64), F32)
    ssq = jnp.zeros((1, 64), F32)
    for k in range(K):
        hn = jnp.maximum((_h1k(xi, xjp, wb, hi, k) - mu) * rs, 0.0)
        h2k = _dot1x(hn, w2_ref[...]) + b2_ref[...]
        ssum = ssum + jnp.sum(h2k, axis=0, keepdims=True)
        ssq = ssq + jnp.sum(h2k * h2k, axis=0, keepdims=True)
    st = jnp.concatenate([ssum, ssq], axis=0)

    @pl.when(g == 0)
    def _():
        st_ref[...] = st

    @pl.when(g != 0)
    def _():
        st_ref[...] = st_ref[...] + st


def _edge3_body(xi_ref, xjp_ref, st1_ref, st2_ref, wa_ref, wb_ref, b1_ref,
                w2_ref, b2_ref, w3_ref, b3_ref, w4b_ref, w4d_ref, b4_ref,
                x1_ref, y_ref, u_ref):
    xi = xi_ref[...]
    xjp = xjp_ref[...]
    wb = wb_ref[...].astype(BF)
    hi = jnp.dot(xi.astype(BF), wa_ref[...].astype(BF),
                 preferred_element_type=F32) + b1_ref[...]
    mu1, rs1 = _norm_consts(st1_ref[...])
    mu2, rs2 = _norm_consts(st2_ref[...])
    acc = jnp.full((TPP, 64), -jnp.inf, F32)
    for k in range(K):
        hn = jnp.maximum((_h1k(xi, xjp, wb, hi, k) - mu1) * rs1, 0.0)
        h2k = _dot1x(hn, w2_ref[...]) + b2_ref[...]
        hn2 = jnp.maximum((h2k - mu2) * rs2, 0.0)
        v = _dot1x(hn2, w3_ref[...]) + b3_ref[...]
        acc = jnp.maximum(acc, v)
    x1_ref[...] = acc
    y_ref[...] = _dot1x(acc, w4b_ref[...])
    u_ref[...] = _dot1x(acc, w4d_ref[...]) + b4_ref[...]


def _pool_body(x1_ref, u_ref, m_ref, w5a_ref, w5b_ref, b5_ref, out_ref):
    p = pl.program_id(1)
    t = (_dot1x(x1_ref[...], w5a_ref[...])
         + _dot1x(u_ref[...] + m_ref[...], w5b_ref[...])
         + b5_ref[...])
    v = jnp.broadcast_to(jnp.max(t, axis=0, keepdims=True), (8, 1024))[None]

    @pl.when(p == 0)
    def _():
        out_ref[...] = v

    @pl.when(p != 0)
    def _():
        out_ref[...] = jnp.maximum(out_ref[...], v)


def _bn_relu_rows(h):
    mu = jnp.mean(h, axis=0, keepdims=True)
    var = jnp.mean((h - mu) ** 2, axis=0, keepdims=True)
    return jnp.maximum((h - mu) * lax.rsqrt(var + EPS), 0.0)


def _head_body(z_ref, w6_ref, b6_ref, w7_ref, b7_ref, w8_ref, b8_ref, o_ref):
    h = _dot1x(z_ref[...], w6_ref[...]) + b6_ref[...]
    h = _bn_relu_rows(h)
    h = _dot1x(h, w7_ref[...]) + b7_ref[...]
    h = _bn_relu_rows(h)
    h = _dot1x(h, w8_ref[...]) + b8_ref[...]
    mx = jnp.max(h, axis=1, keepdims=True)
    e = jnp.exp(h - mx)
    o_ref[...] = h - mx - jnp.log(jnp.sum(e, axis=1, keepdims=True))


def kernel(pos, batch, W1, b1, W2, b2, W3, b3, W4, b4, W5, b5, W6, b6, W7, b7, W8, b8):
    del batch
    posp = jnp.pad(pos, ((0, 0), (0, 13)))
    post = jnp.pad(pos.reshape(B, P, 3).transpose(0, 2, 1),
                   ((0, 0), (0, 13), (0, 0)))
    w1a = jnp.pad(W1[0:3], ((0, 13), (0, 0)))
    w1b = jnp.pad(W1[3:6], ((0, 13), (0, 0)))
    w4a, w4b = W4[:64], W4[64:]
    w4d = w4a - w4b
    w5a, w5b = W5[:64], W5[64:]

    nb = P // TPK

    b1r = b1.reshape(1, 64)
    b2r = b2.reshape(1, 64)

    xjp = pl.pallas_call(
        _knn1_body, grid=(B, nb),
        in_specs=[pl.BlockSpec((TPK, 16), lambda b, p: (b * nb + p, 0)),
                  pl.BlockSpec((1, 16, P), lambda b, p: (b, 0, 0)),
                  pl.BlockSpec((P, 16), lambda b, p: (b, 0))],
        out_specs=pl.BlockSpec((TPK, K * 16), lambda b, p: (b * nb + p, 0)),
        out_shape=jax.ShapeDtypeStruct((NP, K * 16), F32),
    )(posp, post, posp)

    st1 = pl.pallas_call(
        _edge1_body, grid=(NP // TPP,),
        in_specs=[pl.BlockSpec((TPP, 16), lambda p: (p, 0)),
                  pl.BlockSpec((TPP, K * 16), lambda p: (p, 0)),
                  pl.BlockSpec((16, 64), lambda p: (0, 0)),
                  pl.BlockSpec((16, 64), lambda p: (0, 0)),
                  pl.BlockSpec((1, 64), lambda p: (0, 0))],
        out_specs=pl.BlockSpec((2, 64), lambda p: (0, 0)),
        out_shape=jax.ShapeDtypeStruct((2, 64), F32),
    )(posp, xjp, w1a, w1b, b1r)

    st2 = pl.pallas_call(
        _edge2_body, grid=(NP // TPP,),
        in_specs=[pl.BlockSpec((TPP, 16), lambda p: (p, 0)),
                  pl.BlockSpec((TPP, K * 16), lambda p: (p, 0)),
                  pl.BlockSpec((2, 64), lambda p: (0, 0)),
                  pl.BlockSpec((16, 64), lambda p: (0, 0)),
                  pl.BlockSpec((16, 64), lambda p: (0, 0)),
                  pl.BlockSpec((1, 64), lambda p: (0, 0)),
                  pl.BlockSpec((64, 64), lambda p: (0, 0)),
                  pl.BlockSpec((1, 64), lambda p: (0, 0))],
        out_specs=pl.BlockSpec((2, 64), lambda p: (0, 0)),
        out_shape=jax.ShapeDtypeStruct((2, 64), F32),
    )(posp, xjp, st1, w1a, w1b, b1r, W2, b2r)

    x1, y, u = pl.pallas_call(
        _edge3_body, grid=(NP // TPP,),
        in_specs=[pl.BlockSpec((TPP, 16), lambda p: (p, 0)),
                  pl.BlockSpec((TPP, K * 16), lambda p: (p, 0)),
                  pl.BlockSpec((2, 64), lambda p: (0, 0)),
                  pl.BlockSpec((2, 64), lambda p: (0, 0)),
                  pl.BlockSpec((16, 64), lambda p: (0, 0)),
                  pl.BlockSpec((16, 64), lambda p: (0, 0)),
                  pl.BlockSpec((1, 64), lambda p: (0, 0)),
                  pl.BlockSpec((64, 64), lambda p: (0, 0)),
                  pl.BlockSpec((1, 64), lambda p: (0, 0)),
                  pl.BlockSpec((64, 64), lambda p: (0, 0)),
                  pl.BlockSpec((1, 64), lambda p: (0, 0)),
                  pl.BlockSpec((64, 128), lambda p: (0, 0)),
                  pl.BlockSpec((64, 128), lambda p: (0, 0)),
                  pl.BlockSpec((1, 128), lambda p: (0, 0))],
        out_specs=[pl.BlockSpec((TPP, 64), lambda p: (p, 0)),
                   pl.BlockSpec((TPP, 128), lambda p: (p, 0)),
                   pl.BlockSpec((TPP, 128), lambda p: (p, 0))],
        out_shape=[jax.ShapeDtypeStruct((NP, 64), F32),
                   jax.ShapeDtypeStruct((NP, 128), F32),
                   jax.ShapeDtypeStruct((NP, 128), F32)],
    )(posp, xjp, st1, st2, w1a, w1b, b1r, W2, b2r, W3, b3.reshape(1, 64),
      w4b, w4d, b4.reshape(1, 128))

    x1v = x1.reshape(B, P, 64)
    x1t = x1v.transpose(0, 2, 1)
    idx2 = pl.pallas_call(
        _knn2_body, grid=(B, nb),
        in_specs=[pl.BlockSpec((1, TPK, 64), lambda b, p: (b, p, 0)),
                  pl.BlockSpec((1, 64, P), lambda b, p: (b, 0, 0))],
        out_specs=pl.BlockSpec((TPK, K), lambda b, p: (b * nb + p, 0)),
        out_shape=jax.ShapeDtypeStruct((NP, K), jnp.int32),
    )(x1v, x1t)

    m = _sc_gather_max(y, idx2.reshape(-1))

    pooled = pl.pallas_call(
        _pool_body, grid=(B, P // TPP),
        in_specs=[pl.BlockSpec((TPP, 64), lambda b, p: (b * (P // TPP) + p, 0)),
                  pl.BlockSpec((TPP, 128), lambda b, p: (b * (P // TPP) + p, 0)),
                  pl.BlockSpec((TPP, 128), lambda b, p: (b * (P // TPP) + p, 0)),
                  pl.BlockSpec((64, 1024), lambda b, p: (0, 0)),
                  pl.BlockSpec((128, 1024), lambda b, p: (0, 0)),
                  pl.BlockSpec((1, 1024), lambda b, p: (0, 0))],
        out_specs=pl.BlockSpec((1, 8, 1024), lambda b, p: (b, 0, 0)),
        out_shape=jax.ShapeDtypeStruct((B, 8, 1024), F32),
    )(x1, u, m, w5a, w5b, b5.reshape(1, 1024))
    pooled = pooled[:, 0, :]

    out = pl.pallas_call(
        _head_body,
        in_specs=[pl.BlockSpec((B, 1024), lambda: (0, 0)),
                  pl.BlockSpec((1024, 512), lambda: (0, 0)),
                  pl.BlockSpec((1, 512), lambda: (0, 0)),
                  pl.BlockSpec((512, 256), lambda: (0, 0)),
                  pl.BlockSpec((1, 256), lambda: (0, 0)),
                  pl.BlockSpec((256, 40), lambda: (0, 0)),
                  pl.BlockSpec((1, 40), lambda: (0, 0))],
        out_specs=pl.BlockSpec((B, 40), lambda: (0, 0)),
        out_shape=jax.ShapeDtypeStruct((B, 40), F32),
    )(pooled, W6, b6.reshape(1, 512), W7, b7.reshape(1, 256),
      W8, b8.reshape(1, 40))
    return out


# speedup vs baseline: 1.1543x; 1.0665x over previous
import functools

import jax
import jax.numpy as jnp
from jax import lax
from jax.experimental import pallas as pl
from jax.experimental.pallas import tpu as pltpu
from jax.experimental.pallas import tpu_sc as plsc

B = 16
P = 1024
K = 20
NP = B * P
E = NP * K
EPS = 1e-5
F32 = jnp.float32

TPK = 1024
TPP = 1024

_NC, _NS = 2, 16
_NW = _NC * _NS


def _popmin(d2, iota, nkeep):
    n = float(d2.shape[1])
    cols = []
    for _ in range(nkeep):
        m = jnp.min(d2, axis=1, keepdims=True)
        cand = jnp.where(d2 == m, iota, n)
        j = jnp.min(cand, axis=1, keepdims=True)
        cols.append(j.astype(jnp.int32))
        d2 = jnp.where(iota == j, jnp.inf, d2)
    return cols


BF = jnp.bfloat16


def _dot1x(a, b):
    return jnp.dot(a.astype(BF), b.astype(BF), preferred_element_type=F32)


def _knn1_body(posp_ref, post_ref, posb_ref, xjp_ref):
    x = posp_ref[...]
    xt = post_ref[0]
    pb = posb_ref[...]
    hi = pb.astype(BF)
    r1 = pb - hi.astype(F32)
    mid = r1.astype(BF)
    lo = (r1 - mid.astype(F32)).astype(BF)
    tab = jnp.concatenate([hi, mid, lo], axis=1)
    sq_r = jnp.sum(x * x, axis=1, keepdims=True)
    sq_c = jnp.sum(xt * xt, axis=0, keepdims=True)
    d2 = sq_r + sq_c - 2.0 * _dot1x(x, xt)
    iota = lax.broadcasted_iota(jnp.int32, (TPK, P), 1).astype(F32)
    xjs = []
    for _ in range(K):
        m = jnp.min(d2, axis=1, keepdims=True)
        cand = jnp.where(d2 == m, iota, float(P))
        j = jnp.min(cand, axis=1, keepdims=True)
        sel = iota == j
        xq = jnp.dot(sel.astype(BF), tab, preferred_element_type=F32)
        xjs.append(xq[:, 0:16] + xq[:, 16:32] + xq[:, 32:48])
        d2 = jnp.where(sel, jnp.inf, d2)
    xjp_ref[...] = jnp.concatenate(xjs, axis=1)


def _edge1_body(xi_ref, xjp_ref, wa_ref, wb_ref, b1_ref, st_ref):
    g = pl.program_id(0)
    xi = xi_ref[...]
    xjp = xjp_ref[...]
    wb = wb_ref[...].astype(BF)
    hi = jnp.dot(xi.astype(BF), wa_ref[...].astype(BF),
                 preferred_element_type=F32) + b1_ref[...]
    ssum = jnp.zeros((1, 64), F32)
    ssq = jnp.zeros((1, 64), F32)
    for k in range(K):
        h1k = _h1k(xi, xjp, wb, hi, k)
        ssum = ssum + jnp.sum(h1k, axis=0, keepdims=True)
        ssq = ssq + jnp.sum(h1k * h1k, axis=0, keepdims=True)
    st = jnp.concatenate([ssum, ssq], axis=0)

    @pl.when(g == 0)
    def _():
        st_ref[...] = st

    @pl.when(g != 0)
    def _():
        st_ref[...] = st_ref[...] + st


def _knn2_body(x_ref, xt_ref, idx_ref):
    b = pl.program_id(0)
    p = pl.program_id(1)
    x = x_ref[0]
    xt = xt_ref[0]
    sq_r = jnp.sum(x * x, axis=1, keepdims=True)
    sq_c = jnp.sum(xt * xt, axis=0, keepdims=True)
    d2 = sq_r + sq_c - 2.0 * _dot1x(x, xt)
    iota = lax.broadcasted_iota(jnp.int32, (TPK, P), 1).astype(F32)
    cols = _popmin(d2, iota, K)
    idx_ref[...] = jnp.concatenate(cols, axis=1) + b * P


G4 = 4


def _sc_gather_max(y, idx):
    pw = NP // _NW
    ngrp = pw // G4
    gi = G4 * K
    mesh = plsc.VectorSubcoreMesh(core_axis_name="c", subcore_axis_name="s")

    @functools.partial(
        pl.kernel, mesh=mesh,
        out_type=jax.ShapeDtypeStruct((NP, 128), F32),
        scratch_types=[pltpu.VMEM((pw * K,), jnp.int32),
                       pltpu.VMEM((gi, 128), F32),
                       pltpu.VMEM((gi, 128), F32),
                       pltpu.VMEM((G4, 128), F32),
                       pltpu.SemaphoreType.DMA,
                       pltpu.SemaphoreType.DMA],
    )
    def run(y_hbm, idx_hbm, out_hbm, idx_all, rows0, rows1, out_v, sem0, sem1):
        wid = lax.axis_index("s") * _NC + lax.axis_index("c")
        base = wid * pw
        pltpu.sync_copy(idx_hbm.at[pl.ds(base * K, pw * K)], idx_all)
        pltpu.async_copy(y_hbm.at[idx_all.at[pl.ds(0, gi)]], rows0, sem0)
        pltpu.async_copy(y_hbm.at[idx_all.at[pl.ds(gi, gi)]], rows1, sem1)

        def half(g, rows_v, sem):
            pltpu.make_async_copy(y_hbm.at[idx_all.at[pl.ds(0, gi)]],
                                  rows_v, sem).wait()
            for i in range(G4):
                for c in range(8):
                    v = rows_v[i * K, pl.ds(c * 16, 16)]
                    for r in range(1, K):
                        v = jnp.maximum(v, rows_v[i * K + r, pl.ds(c * 16, 16)])
                    out_v[i, pl.ds(c * 16, 16)] = v
            pltpu.sync_copy(out_v, out_hbm.at[pl.ds(base + g * G4, G4)])
            nxt = g + 2

            @pl.when(nxt < ngrp)
            def _():
                pltpu.async_copy(y_hbm.at[idx_all.at[pl.ds(nxt * gi, gi)]],
                                 rows_v, sem)

        def body(gg, carry):
            half(2 * gg, rows0, sem0)
            half(2 * gg + 1, rows1, sem1)
            return carry

        lax.fori_loop(0, ngrp // 2, body, 0)

    return run(y, idx)


def _stats_update(st_ref, h, g):
    st = jnp.concatenate([jnp.sum(h, axis=0, keepdims=True),
                          jnp.sum(h * h, axis=0, keepdims=True)], axis=0)

    @pl.when(g == 0)
    def _():
        st_ref[...] = st

    @pl.when(g != 0)
    def _():
        st_ref[...] = st_ref[...] + st


def _norm_consts(st):
    mu = st[0:1] * (1.0 / E)
    var = st[1:2] * (1.0 / E) - mu * mu
    return mu, lax.rsqrt(var + EPS)


def _h1k(xi, xjp, wb, hi, k):
    xj = xjp[:, k * 16:(k + 1) * 16]
    return hi + jnp.dot((xj - xi).astype(BF), wb, preferred_element_type=F32)


def _edge2_body(xi_ref, xjp_ref, st1_ref, wa_ref, wb_ref, b1_ref,
                w2_ref, b2_ref, st_ref):
    g = pl.program_id(0)
    xi = xi_ref[...]
    xjp = xjp_ref[...]
    wb = wb_ref[...].astype(BF)
    hi = jnp.dot(xi.astype(BF), wa_ref[...].astype(BF),
                 preferred_element_type=F32) + b1_ref[...]
    mu, rs = _norm_consts(st1_ref[...])
    ssum = jnp.zeros((1, 64), F32)
    ssq = jnp.zeros((1, 64), F32)
    for k in range(K):
        hn = jnp.maximum((_h1k(xi, xjp, wb, hi, k) - mu) * rs, 0.0)
        h2k = _dot1x(hn, w2_ref[...]) + b2_ref[...]
        ssum = ssum + jnp.sum(h2k, axis=0, keepdims=True)
        ssq = ssq + jnp.sum(h2k * h2k, axis=0, keepdims=True)
    st = jnp.concatenate([ssum, ssq], axis=0)

    @pl.when(g == 0)
    def _():
        st_ref[...] = st

    @pl.when(g != 0)
    def _():
        st_ref[...] = st_ref[...] + st


def _edge3_body(xi_ref, xjp_ref, st1_ref, st2_ref, wa_ref, wb_ref, b1_ref,
                w2_ref, b2_ref, w3_ref, b3_ref, w4b_ref, w4d_ref, b4_ref,
                x1_ref, y_ref, u_ref):
    xi = xi_ref[...]
    xjp = xjp_ref[...]
    wb = wb_ref[...].astype(BF)
    hi = jnp.dot(xi.astype(BF), wa_ref[...].astype(BF),
                 preferred_element_type=F32) + b1_ref[...]
    mu1, rs1 = _norm_consts(st1_ref[...])
    mu2, rs2 = _norm_consts(st2_ref[...])
    acc = jnp.full((TPP, 64), -jnp.inf, F32)
    for k in range(K):
        hn = jnp.maximum((_h1k(xi, xjp, wb, hi, k) - mu1) * rs1, 0.0)
        h2k = _dot1x(hn, w2_ref[...]) + b2_ref[...]
        hn2 = jnp.maximum((h2k - mu2) * rs2, 0.0)
        v = _dot1x(hn2, w3_ref[...]) + b3_ref[...]
        acc = jnp.maximum(acc, v)
    x1_ref[...] = acc
    y_ref[...] = _dot1x(acc, w4b_ref[...])
    u_ref[...] = _dot1x(acc, w4d_ref[...]) + b4_ref[...]


def _pool_body(x1_ref, u_ref, m_ref, w5a_ref, w5b_ref, b5_ref, out_ref):
    p = pl.program_id(1)
    t = (_dot1x(x1_ref[...], w5a_ref[...])
         + _dot1x(u_ref[...] + m_ref[...], w5b_ref[...])
         + b5_ref[...])
    v = jnp.broadcast_to(jnp.max(t, axis=0, keepdims=True), (8, 1024))[None]

    @pl.when(p == 0)
    def _():
        out_ref[...] = v

    @pl.when(p != 0)
    def _():
        out_ref[...] = jnp.maximum(out_ref[...], v)


def _bn_relu_rows(h):
    mu = jnp.mean(h, axis=0, keepdims=True)
    var = jnp.mean((h - mu) ** 2, axis=0, keepdims=True)
    return jnp.maximum((h - mu) * lax.rsqrt(var + EPS), 0.0)


def _head_body(z_ref, w6_ref, b6_ref, w7_ref, b7_ref, w8_ref, b8_ref, o_ref):
    h = _dot1x(z_ref[...], w6_ref[...]) + b6_ref[...]
    h = _bn_relu_rows(h)
    h = _dot1x(h, w7_ref[...]) + b7_ref[...]
    h = _bn_relu_rows(h)
    h = _dot1x(h, w8_ref[...]) + b8_ref[...]
    mx = jnp.max(h, axis=1, keepdims=True)
    e = jnp.exp(h - mx)
    o_ref[...] = h - mx - jnp.log(jnp.sum(e, axis=1, keepdims=True))


def kernel(pos, batch, W1, b1, W2, b2, W3, b3, W4, b4, W5, b5, W6, b6, W7, b7, W8, b8):
    del batch
    posp = jnp.pad(pos, ((0, 0), (0, 13)))
    post = jnp.pad(pos.reshape(B, P, 3).transpose(0, 2, 1),
                   ((0, 0), (0, 13), (0, 0)))
    w1a = jnp.pad(W1[0:3], ((0, 13), (0, 0)))
    w1b = jnp.pad(W1[3:6], ((0, 13), (0, 0)))
    w4a, w4b = W4[:64], W4[64:]
    w4d = w4a - w4b
    w5a, w5b = W5[:64], W5[64:]

    nb = P // TPK

    b1r = b1.reshape(1, 64)
    b2r = b2.reshape(1, 64)

    xjp = pl.pallas_call(
        _knn1_body, grid=(B, nb),
        in_specs=[pl.BlockSpec((TPK, 16), lambda b, p: (b * nb + p, 0)),
                  pl.BlockSpec((1, 16, P), lambda b, p: (b, 0, 0)),
                  pl.BlockSpec((P, 16), lambda b, p: (b, 0))],
        out_specs=pl.BlockSpec((TPK, K * 16), lambda b, p: (b * nb + p, 0)),
        out_shape=jax.ShapeDtypeStruct((NP, K * 16), F32),
    )(posp, post, posp)

    st1 = pl.pallas_call(
        _edge1_body, grid=(NP // TPP,),
        in_specs=[pl.BlockSpec((TPP, 16), lambda p: (p, 0)),
                  pl.BlockSpec((TPP, K * 16), lambda p: (p, 0)),
                  pl.BlockSpec((16, 64), lambda p: (0, 0)),
                  pl.BlockSpec((16, 64), lambda p: (0, 0)),
                  pl.BlockSpec((1, 64), lambda p: (0, 0))],
        out_specs=pl.BlockSpec((2, 64), lambda p: (0, 0)),
        out_shape=jax.ShapeDtypeStruct((2, 64), F32),
    )(posp, xjp, w1a, w1b, b1r)

    st2 = pl.pallas_call(
        _edge2_body, grid=(NP // TPP,),
        in_specs=[pl.BlockSpec((TPP, 16), lambda p: (p, 0)),
                  pl.BlockSpec((TPP, K * 16), lambda p: (p, 0)),
                  pl.BlockSpec((2, 64), lambda p: (0, 0)),
                  pl.BlockSpec((16, 64), lambda p: (0, 0)),
                  pl.BlockSpec((16, 64), lambda p: (0, 0)),
                  pl.BlockSpec((1, 64), lambda p: (0, 0)),
                  pl.BlockSpec((64, 64), lambda p: (0, 0)),
                  pl.BlockSpec((1, 64), lambda p: (0, 0))],
        out_specs=pl.BlockSpec((2, 64), lambda p: (0, 0)),
        out_shape=jax.ShapeDtypeStruct((2, 64), F32),
    )(posp, xjp, st1, w1a, w1b, b1r, W2, b2r)

    x1, y, u = pl.pallas_call(
        _edge3_body, grid=(NP // TPP,),
        in_specs=[pl.BlockSpec((TPP, 16), lambda p: (p, 0)),
                  pl.BlockSpec((TPP, K * 16), lambda p: (p, 0)),
                  pl.BlockSpec((2, 64), lambda p: (0, 0)),
                  pl.BlockSpec((2, 64), lambda p: (0, 0)),
                  pl.BlockSpec((16, 64), lambda p: (0, 0)),
                  pl.BlockSpec((16, 64), lambda p: (0, 0)),
                  pl.BlockSpec((1, 64), lambda p: (0, 0)),
                  pl.BlockSpec((64, 64), lambda p: (0, 0)),
                  pl.BlockSpec((1, 64), lambda p: (0, 0)),
                  pl.BlockSpec((64, 64), lambda p: (0, 0)),
                  pl.BlockSpec((1, 64), lambda p: (0, 0)),
                  pl.BlockSpec((64, 128), lambda p: (0, 0)),
                  pl.BlockSpec((64, 128), lambda p: (0, 0)),
                  pl.BlockSpec((1, 128), lambda p: (0, 0))],
        out_specs=[pl.BlockSpec((TPP, 64), lambda p: (p, 0)),
                   pl.BlockSpec((TPP, 128), lambda p: (p, 0)),
                   pl.BlockSpec((TPP, 128), lambda p: (p, 0))],
        out_shape=[jax.ShapeDtypeStruct((NP, 64), F32),
                   jax.ShapeDtypeStruct((NP, 128), F32),
                   jax.ShapeDtypeStruct((NP, 128), F32)],
    )(posp, xjp, st1, st2, w1a, w1b, b1r, W2, b2r, W3, b3.reshape(1, 64),
      w4b, w4d, b4.reshape(1, 128))

    x1v = x1.reshape(B, P, 64)
    x1t = x1v.transpose(0, 2, 1)
    idx2 = pl.pallas_call(
        _knn2_body, grid=(B, nb),
        in_specs=[pl.BlockSpec((1, TPK, 64), lambda b, p: (b, p, 0)),
                  pl.BlockSpec((1, 64, P), lambda b, p: (b, 0, 0))],
        out_specs=pl.BlockSpec((TPK, K), lambda b, p: (b * nb + p, 0)),
        out_shape=jax.ShapeDtypeStruct((NP, K), jnp.int32),
    )(x1v, x1t)

    m = _sc_gather_max(y, idx2.reshape(-1))

    pooled = pl.pallas_call(
        _pool_body, grid=(B, P // TPP),
        in_specs=[pl.BlockSpec((TPP, 64), lambda b, p: (b * (P // TPP) + p, 0)),
                  pl.BlockSpec((TPP, 128), lambda b, p: (b * (P // TPP) + p, 0)),
                  pl.BlockSpec((TPP, 128), lambda b, p: (b * (P // TPP) + p, 0)),
                  pl.BlockSpec((64, 1024), lambda b, p: (0, 0)),
                  pl.BlockSpec((128, 1024), lambda b, p: (0, 0)),
                  pl.BlockSpec((1, 1024), lambda b, p: (0, 0))],
        out_specs=pl.BlockSpec((1, 8, 1024), lambda b, p: (b, 0, 0)),
        out_shape=jax.ShapeDtypeStruct((B, 8, 1024), F32),
    )(x1, u, m, w5a, w5b, b5.reshape(1, 1024))
    pooled = pooled[:, 0, :]

    out = pl.pallas_call(
        _head_body,
        in_specs=[pl.BlockSpec((B, 1024), lambda: (0, 0)),
                  pl.BlockSpec((1024, 512), lambda: (0, 0)),
                  pl.BlockSpec((1, 512), lambda: (0, 0)),
                  pl.BlockSpec((512, 256), lambda: (0, 0)),
                  pl.BlockSpec((1, 256), lambda: (0, 0)),
                  pl.BlockSpec((256, 40), lambda: (0, 0)),
                  pl.BlockSpec((1, 40), lambda: (0, 0))],
        out_specs=pl.BlockSpec((B, 40), lambda: (0, 0)),
        out_shape=jax.ShapeDtypeStruct((B, 40), F32),
    )(pooled, W6, b6.reshape(1, 512), W7, b7.reshape(1, 256),
      W8, b8.reshape(1, 40))
    return out
